# Initial kernel scaffold; baseline (speedup 1.0000x reference)
import jax
import jax.numpy as jnp
from jax.experimental import pallas as pl


def kernel(x, edge_index, W1, b1, Wg0, bg0, Wg1, bg1, Wc1, bc1, Wc2, bc2):
    raise NotImplementedError("write your pallas kernel here")


# speedup vs baseline: 10.5209x; 10.5209x over previous
import functools

import jax
import jax.numpy as jnp
from jax import lax
from jax.experimental import pallas as pl
from jax.experimental.pallas import tpu as pltpu
from jax.experimental.pallas import tpu_sc as plsc

N = 10000
E = 320000
F = 128
N_PAD = 10240
NB = N_PAD // F
NW = 32
EPT = 10112
CHUNKS = EPT // F
E_PAD = NW * EPT
DUMMY = N

_mesh = plsc.VectorSubcoreMesh(core_axis_name="c", subcore_axis_name="s")


DW = 16


@functools.partial(
    pl.kernel,
    out_type=jax.ShapeDtypeStruct((2, N_PAD, DW), jnp.float32),
    mesh=_mesh,
    scratch_types=[
        pltpu.VMEM((CHUNKS, F), jnp.int32),
        pltpu.VMEM((F, DW), jnp.float32),
        pltpu.VMEM((F, DW), jnp.float32),
        pltpu.VMEM_SHARED((N_PAD, DW), jnp.float32),
    ],
)
def _deg_kernel(dst_hbm, out_hbm, dst_v, ones_v, zero_v, deg_sh):
    c = lax.axis_index("c")
    s = lax.axis_index("s")
    wid = c * 16 + s
    rows_per_tile = N_PAD // 16
    z16 = jnp.zeros((16,), jnp.float32)
    o16 = jnp.full((16,), 1.0, jnp.float32)

    def fill(i, carry):
        ones_v[i, pl.ds(0, 16)] = o16
        zero_v[i, pl.ds(0, 16)] = z16
        return carry

    lax.fori_loop(0, F, fill, 0)

    def zsh(k, carry):
        pltpu.sync_copy(zero_v, deg_sh.at[pl.ds(s * rows_per_tile + k * F, F)])
        return carry

    lax.fori_loop(0, rows_per_tile // F, zsh, 0)
    pltpu.sync_copy(dst_hbm.at[wid], dst_v)
    plsc.subcore_barrier()

    def body(j, carry):
        pltpu.sync_copy(ones_v, deg_sh.at[dst_v.at[j]], add=True)
        return carry

    lax.fori_loop(0, CHUNKS, body, 0)
    plsc.subcore_barrier()
    pltpu.sync_copy(deg_sh.at[pl.ds(s * rows_per_tile, rows_per_tile)],
                    out_hbm.at[c, pl.ds(s * rows_per_tile, rows_per_tile)])


@functools.partial(
    pl.kernel,
    out_type=jax.ShapeDtypeStruct((2, N_PAD, F), jnp.float32),
    mesh=_mesh,
    scratch_types=[
        pltpu.VMEM((CHUNKS, F), jnp.int32),
        pltpu.VMEM((CHUNKS, F), jnp.int32),
        pltpu.VMEM((F, F), jnp.float32),
        pltpu.VMEM_SHARED((N_PAD, F), jnp.float32),
        pltpu.SemaphoreType.DMA,
    ],
)
def _agg_kernel(table_hbm, src_hbm, dst_hbm, out_hbm,
                src_v, dst_v, rows_v, acc_sh, sem):
    c = lax.axis_index("c")
    s = lax.axis_index("s")
    wid = c * 16 + s
    z16 = jnp.zeros((16,), jnp.float32)

    def zbody(i, carry):
        for k in range(8):
            rows_v[i, pl.ds(k * 16, 16)] = z16
        return carry

    lax.fori_loop(0, F, zbody, 0)
    rows_per_tile = N_PAD // 16

    def zsh(k, carry):
        pltpu.sync_copy(rows_v, acc_sh.at[pl.ds(s * rows_per_tile + k * F, F)])
        return carry

    lax.fori_loop(0, rows_per_tile // F, zsh, 0)
    pltpu.sync_copy(src_hbm.at[wid], src_v)
    pltpu.sync_copy(dst_hbm.at[wid], dst_v)
    plsc.subcore_barrier()

    def body(j, carry):
        pltpu.async_copy(table_hbm.at[src_v.at[j]], rows_v, sem).wait()
        pltpu.sync_copy(rows_v, acc_sh.at[dst_v.at[j]], add=True)
        return carry

    lax.fori_loop(0, CHUNKS, body, 0)
    plsc.subcore_barrier()
    pltpu.sync_copy(acc_sh.at[pl.ds(s * rows_per_tile, rows_per_tile)],
                    out_hbm.at[c, pl.ds(s * rows_per_tile, rows_per_tile)])


def _diag(dinv_row):
    row = lax.broadcasted_iota(jnp.int32, (F, F), 0)
    col = lax.broadcasted_iota(jnp.int32, (F, F), 1)
    return jnp.where(row == col, jnp.broadcast_to(dinv_row, (F, F)), 0.0)


def _mm(a, b):
    return jax.lax.dot_general(a, b, (((1,), (0,)), ((), ())),
                               preferred_element_type=jnp.float32)


def _tc1_body(x, dega, degb, w1, b1, wg0, bg0, dinv_o, h0_o, gs0_o):
    dinv = lax.rsqrt(dega[...] + degb[...] + 1.0)
    dinv_o[...] = dinv
    h0 = _mm(x[...], w1[...]) + b1[...]
    h0_o[...] = h0
    g0 = _mm(h0, wg0[...]) + bg0[...]
    gs0_o[...] = _mm(_diag(dinv[0]), g0)


def _tc2_body(s0a, s0b, gs0, h0, dinv, wg1, bg1, h01_o, gs1_o):
    d = _diag(dinv[...][0])
    h1 = jax.nn.relu(_mm(d, s0a[...] + s0b[...] + gs0[...]))
    h01 = h0[...] + h1
    h01_o[...] = h01
    g1 = _mm(h01, wg1[...]) + bg1[...]
    gs1_o[...] = _mm(d, g1)


def _tc3_body(s1a, s1b, gs1, h01, dinv, wc1, bc1, wc2, bc2, out_o):
    d = _diag(dinv[...][0])
    h2 = jax.nn.relu(_mm(d, s1a[...] + s1b[...] + gs1[...]))
    f = h01[...] + h2
    r = jax.nn.relu(_mm(f, wc1[...]) + bc1[...])
    out_o[...] = _mm(r, wc2[...]) + bc2[...]


_blk = pl.BlockSpec((F, F), lambda i: (i, 0))
_row = pl.BlockSpec((1, 1, F), lambda i: (i, 0, 0))
_w = pl.BlockSpec((F, F), lambda i: (0, 0))
_b = pl.BlockSpec((1, F), lambda i: (0, 0))

_nf32 = jax.ShapeDtypeStruct((N_PAD, F), jnp.float32)

_tc1 = pl.pallas_call(
    _tc1_body, grid=(NB,),
    in_specs=[_blk, _row, _row, _w, _b, _w, _b],
    out_specs=[_row, _blk, _blk],
    out_shape=[jax.ShapeDtypeStruct((NB, 1, F), jnp.float32), _nf32, _nf32],
)

_tc2 = pl.pallas_call(
    _tc2_body, grid=(NB,),
    in_specs=[_blk, _blk, _blk, _blk, _row, _w, _b],
    out_specs=[_blk, _blk],
    out_shape=[_nf32, _nf32],
)

_tc3 = pl.pallas_call(
    _tc3_body, grid=(NB,),
    in_specs=[_blk, _blk, _blk, _blk, _row, _w, _b, _w, _b],
    out_specs=[_blk],
    out_shape=[_nf32],
)


def kernel(x, edge_index, W1, b1, Wg0, bg0, Wg1, bg1, Wc1, bc1, Wc2, bc2):
    src = edge_index[0]
    dst = edge_index[1]
    pad = E_PAD - E
    src_r = jnp.concatenate(
        [src, jnp.full((pad,), DUMMY, jnp.int32)]).reshape(NW, CHUNKS, F)
    dst_r = jnp.concatenate(
        [dst, jnp.full((pad,), DUMMY, jnp.int32)]).reshape(NW, CHUNKS, F)
    x_p = jnp.zeros((N_PAD, F), jnp.float32).at[:N].set(x)

    deg = _deg_kernel(dst_r)
    b1r = b1.reshape(1, F)
    bg0r = bg0.reshape(1, F)
    bg1r = bg1.reshape(1, F)
    bc1r = bc1.reshape(1, F)
    bc2r = bc2.reshape(1, F)
    dega = deg[0, :, 0].reshape(NB, 1, F)
    degb = deg[1, :, 0].reshape(NB, 1, F)
    dinv, h0, gs0 = _tc1(x_p, dega, degb, W1, b1r, Wg0, bg0r)
    s0 = _agg_kernel(gs0, src_r, dst_r)
    h01, gs1 = _tc2(s0[0], s0[1], gs0, h0, dinv, Wg1, bg1r)
    s1 = _agg_kernel(gs1, src_r, dst_r)
    out, = _tc3(s1[0], s1[1], gs1, h01, dinv, Wc1, bc1r, Wc2, bc2r)
    return out[:N]
